# baseline (speedup 1.0000x reference)
import jax
import jax.numpy as jnp
from jax import lax
from jax.experimental import pallas as pl
from jax.experimental.pallas import tpu as pltpu

_KERNEL = 25
_PAD = (_KERNEL - 1) // 2
_E = 8
_B, _L, _V = 4, 2048, 256
_BV = _B * _V
_P = 1024
_P_BLK = 1024


def _top2_coeffs(probs):
    iota = jax.lax.broadcasted_iota(jnp.int32, probs.shape, 0)
    m1 = jnp.max(probs, axis=0, keepdims=True)
    i1 = jnp.min(jnp.where(probs == m1, iota, _E), axis=0, keepdims=True)
    mask1 = iota == i1
    p2 = jnp.where(mask1, -jnp.inf, probs)
    m2 = jnp.max(p2, axis=0, keepdims=True)
    i2 = jnp.min(jnp.where(p2 == m2, iota, _E), axis=0, keepdims=True)
    mask2 = iota == i2
    return probs * (mask1.astype(probs.dtype) + mask2.astype(probs.dtype))


def _softmax0(logits):
    z = logits - jnp.max(logits, axis=0, keepdims=True)
    ez = jnp.exp(z)
    return ez / jnp.sum(ez, axis=0, keepdims=True)


def _fused_kernel(x_ref, gws_ref, gwt_ref, ebs_ref, ebt_ref,
                  ews_ref, ewt_ref,
                  out_ref, ptm_ref,
                  sea_sc, trend_sc, cs_sc, ct_sc):
    e = pl.program_id(0)
    pb = pl.program_id(1)

    @pl.when((e == 0) & (pb == 0))
    def _prologue():
        ptm = jnp.zeros((_E, _V), jnp.float32)
        for b in range(_B):
            x = x_ref[b]
            front = jnp.broadcast_to(x[0:1, :], (_PAD, _V))
            back = jnp.broadcast_to(x[_L - 1:_L, :], (_PAD, _V))
            xp = jnp.concatenate([front, x, back], axis=0)
            acc = xp[0:_L, :]
            for k in range(1, _KERNEL):
                acc = acc + xp[k:k + _L, :]
            mov = acc * (1.0 / _KERNEL)
            sea = x - mov

            logits_s = jnp.dot(gws_ref[...], sea,
                               preferred_element_type=jnp.float32)
            logits_t = jnp.dot(gwt_ref[...], mov,
                               preferred_element_type=jnp.float32)
            probs_s = _softmax0(logits_s)
            probs_t = _softmax0(logits_t)

            col = slice(b * _V, (b + 1) * _V)
            cs_sc[:, col] = _top2_coeffs(probs_s)
            ct_sc[:, col] = _top2_coeffs(probs_t)
            sea_sc[:, col] = sea.astype(jnp.bfloat16)
            trend_sc[:, col] = mov.astype(jnp.bfloat16)
            ptm = ptm + probs_t * (1.0 / _B)
        ptm_ref[...] = ptm

    ws = ews_ref[0].astype(jnp.bfloat16)
    wt = ewt_ref[0].astype(jnp.bfloat16)
    ys = jnp.dot(ws, sea_sc[...], preferred_element_type=jnp.float32)
    yt = jnp.dot(wt, trend_sc[...], preferred_element_type=jnp.float32)

    cs_row = cs_sc[e, :][None, :]
    ct_row = ct_sc[e, :][None, :]
    ebs_col = ebs_ref[e, pl.ds(pb * _P_BLK, _P_BLK)][:, None]
    ebt_col = ebt_ref[e, pl.ds(pb * _P_BLK, _P_BLK)][:, None]
    contrib = cs_row * (ys + ebs_col) + ct_row * (yt + ebt_col)

    @pl.when(e == 0)
    def _():
        for b in range(_B):
            out_ref[b, pl.ds(pb * _P_BLK, _P_BLK), :] = contrib[:, b * _V:(b + 1) * _V]

    @pl.when(e > 0)
    def _():
        for b in range(_B):
            out_ref[b, pl.ds(pb * _P_BLK, _P_BLK), :] += contrib[:, b * _V:(b + 1) * _V]


@jax.jit
def kernel(x, Gw_sea, Ew_sea, Eb_sea, Gw_trend, Ew_trend, Eb_trend):
    out, ptm = pl.pallas_call(
        _fused_kernel,
        grid=(_E, _P // _P_BLK),
        in_specs=[
            pl.BlockSpec((_B, _L, _V), lambda e, pb: (0, 0, 0)),
            pl.BlockSpec((_E, _L), lambda e, pb: (0, 0)),
            pl.BlockSpec((_E, _L), lambda e, pb: (0, 0)),
            pl.BlockSpec((_E, _P), lambda e, pb: (0, 0)),
            pl.BlockSpec((_E, _P), lambda e, pb: (0, 0)),
            pl.BlockSpec((1, _P_BLK, _L), lambda e, pb: (e, pb, 0)),
            pl.BlockSpec((1, _P_BLK, _L), lambda e, pb: (e, pb, 0)),
        ],
        out_specs=[
            pl.BlockSpec((_B, _P, _V), lambda e, pb: (0, 0, 0)),
            pl.BlockSpec((_E, _V), lambda e, pb: (0, 0)),
        ],
        out_shape=[
            jax.ShapeDtypeStruct((_B, _P, _V), jnp.float32),
            jax.ShapeDtypeStruct((_E, _V), jnp.float32),
        ],
        scratch_shapes=[
            pltpu.VMEM((_L, _BV), jnp.bfloat16),
            pltpu.VMEM((_L, _BV), jnp.bfloat16),
            pltpu.VMEM((_E, _BV), jnp.float32),
            pltpu.VMEM((_E, _BV), jnp.float32),
        ],
        compiler_params=pltpu.CompilerParams(
            dimension_semantics=("arbitrary", "arbitrary"),
            vmem_limit_bytes=100 * 1024 * 1024,
        ),
    )(x, Gw_sea, Gw_trend, Eb_sea, Eb_trend, Ew_sea, Ew_trend)

    return out, jnp.transpose(ptm, (1, 0))


# speedup vs baseline: 1.1055x; 1.1055x over previous
import jax
import jax.numpy as jnp
from jax import lax
from jax.experimental import pallas as pl
from jax.experimental.pallas import tpu as pltpu

_KERNEL = 25
_PAD = (_KERNEL - 1) // 2
_E = 8
_B, _L, _V = 4, 2048, 256
_BV = _B * _V
_P = 1024
_P_BLK = 512


def _top2_coeffs(probs):
    iota = jax.lax.broadcasted_iota(jnp.int32, probs.shape, 0)
    m1 = jnp.max(probs, axis=0, keepdims=True)
    i1 = jnp.min(jnp.where(probs == m1, iota, _E), axis=0, keepdims=True)
    mask1 = iota == i1
    p2 = jnp.where(mask1, -jnp.inf, probs)
    m2 = jnp.max(p2, axis=0, keepdims=True)
    i2 = jnp.min(jnp.where(p2 == m2, iota, _E), axis=0, keepdims=True)
    mask2 = iota == i2
    return probs * (mask1.astype(probs.dtype) + mask2.astype(probs.dtype))


def _softmax0(logits):
    z = logits - jnp.max(logits, axis=0, keepdims=True)
    ez = jnp.exp(z)
    return ez / jnp.sum(ez, axis=0, keepdims=True)


def _fused_kernel(x_ref, gws_ref, gwt_ref, ebs_ref, ebt_ref,
                  ews_ref, ewt_ref,
                  out_ref, ptm_ref,
                  sea_sc, trend_sc, cs_sc, ct_sc):
    e = pl.program_id(0)
    pb = pl.program_id(1)

    @pl.when((e == 0) & (pb == 0))
    def _prologue():
        ptm = jnp.zeros((_E, _V), jnp.float32)
        for b in range(_B):
            x = x_ref[b]
            front = jnp.broadcast_to(x[0:1, :], (_PAD, _V))
            back = jnp.broadcast_to(x[_L - 1:_L, :], (_PAD, _V))
            xp = jnp.concatenate([front, x, back], axis=0)
            a2 = xp[0:2071] + xp[1:2072]
            a4 = a2[0:2069] + a2[2:2071]
            a8 = a4[0:2065] + a4[4:2069]
            a16 = a8[0:2057] + a8[8:2065]
            a24 = a16[0:2049] + a8[16:2065]
            a25 = a24[0:_L] + xp[24:24 + _L]
            mov = a25 * (1.0 / _KERNEL)
            sea = x - mov

            logits_s = jnp.dot(gws_ref[...], sea,
                               preferred_element_type=jnp.float32)
            logits_t = jnp.dot(gwt_ref[...], mov,
                               preferred_element_type=jnp.float32)
            probs_s = _softmax0(logits_s)
            probs_t = _softmax0(logits_t)

            col = slice(b * _V, (b + 1) * _V)
            cs_sc[:, col] = _top2_coeffs(probs_s)
            ct_sc[:, col] = _top2_coeffs(probs_t)
            sea_sc[:, col] = sea.astype(jnp.bfloat16)
            trend_sc[:, col] = mov.astype(jnp.bfloat16)
            ptm = ptm + probs_t * (1.0 / _B)
        ptm_ref[...] = ptm

    ws = ews_ref[0].astype(jnp.bfloat16)
    wt = ewt_ref[0].astype(jnp.bfloat16)
    ys = jnp.dot(ws, sea_sc[...], preferred_element_type=jnp.float32)
    yt = jnp.dot(wt, trend_sc[...], preferred_element_type=jnp.float32)

    cs_row = cs_sc[e, :][None, :]
    ct_row = ct_sc[e, :][None, :]
    ebs_col = ebs_ref[e, pl.ds(pb * _P_BLK, _P_BLK)][:, None]
    ebt_col = ebt_ref[e, pl.ds(pb * _P_BLK, _P_BLK)][:, None]
    contrib = cs_row * (ys + ebs_col) + ct_row * (yt + ebt_col)

    @pl.when(e == 0)
    def _():
        for b in range(_B):
            out_ref[b, pl.ds(pb * _P_BLK, _P_BLK), :] = contrib[:, b * _V:(b + 1) * _V]

    @pl.when(e > 0)
    def _():
        for b in range(_B):
            out_ref[b, pl.ds(pb * _P_BLK, _P_BLK), :] += contrib[:, b * _V:(b + 1) * _V]


@jax.jit
def kernel(x, Gw_sea, Ew_sea, Eb_sea, Gw_trend, Ew_trend, Eb_trend):
    out, ptm = pl.pallas_call(
        _fused_kernel,
        grid=(_E, _P // _P_BLK),
        in_specs=[
            pl.BlockSpec((_B, _L, _V), lambda e, pb: (0, 0, 0)),
            pl.BlockSpec((_E, _L), lambda e, pb: (0, 0)),
            pl.BlockSpec((_E, _L), lambda e, pb: (0, 0)),
            pl.BlockSpec((_E, _P), lambda e, pb: (0, 0)),
            pl.BlockSpec((_E, _P), lambda e, pb: (0, 0)),
            pl.BlockSpec((1, _P_BLK, _L), lambda e, pb: (e, pb, 0)),
            pl.BlockSpec((1, _P_BLK, _L), lambda e, pb: (e, pb, 0)),
        ],
        out_specs=[
            pl.BlockSpec((_B, _P, _V), lambda e, pb: (0, 0, 0)),
            pl.BlockSpec((_E, _V), lambda e, pb: (0, 0)),
        ],
        out_shape=[
            jax.ShapeDtypeStruct((_B, _P, _V), jnp.float32),
            jax.ShapeDtypeStruct((_E, _V), jnp.float32),
        ],
        scratch_shapes=[
            pltpu.VMEM((_L, _BV), jnp.bfloat16),
            pltpu.VMEM((_L, _BV), jnp.bfloat16),
            pltpu.VMEM((_E, _BV), jnp.float32),
            pltpu.VMEM((_E, _BV), jnp.float32),
        ],
        compiler_params=pltpu.CompilerParams(
            dimension_semantics=("arbitrary", "arbitrary"),
            vmem_limit_bytes=100 * 1024 * 1024,
        ),
    )(x, Gw_sea, Gw_trend, Eb_sea, Eb_trend, Ew_sea, Ew_trend)

    return out, jnp.transpose(ptm, (1, 0))
